# Initial kernel scaffold; baseline (speedup 1.0000x reference)
import jax
import jax.numpy as jnp
from jax.experimental import pallas as pl


def kernel(x, coords):
    raise NotImplementedError("write your pallas kernel here")


# speedup vs baseline: 3.3004x; 3.3004x over previous
import functools

import jax
import jax.numpy as jnp
from jax import lax
from jax.experimental import pallas as pl
from jax.experimental.pallas import tpu as pltpu
from jax.experimental.pallas import tpu_sc as plsc

NH = 64
L = 16
NC, NS = 2, 16
NW = NC * NS
GROUP = 8


def _ce(ka, va, kb, vb):
    c = ka <= kb
    lo_k = jnp.where(c, ka, kb)
    lo_v = jnp.where(c, va, vb)
    hi_k = jnp.where(c, kb, ka)
    hi_v = jnp.where(c, vb, va)
    return lo_k, lo_v, hi_k, hi_v


def _merge_chunk(ks, vs, ck, cv):
    k0, k1, k2, k3 = ks
    v0, v1, v2, v3 = vs
    rk = lax.rev(ck, (0,))
    rv = lax.rev(cv, (0,))
    k3, v3, _, _ = _ce(k3, v3, rk, rv)
    k0, v0, k2, v2 = _ce(k0, v0, k2, v2)
    k1, v1, k3, v3 = _ce(k1, v1, k3, v3)
    k0, v0, k1, v1 = _ce(k0, v0, k1, v1)
    k2, v2, k3, v3 = _ce(k2, v2, k3, v3)
    k0, v0 = plsc.sort_key_val(k0, v0)
    k1, v1 = plsc.sort_key_val(k1, v1)
    k2, v2 = plsc.sort_key_val(k2, v2)
    k3, v3 = plsc.sort_key_val(k3, v3)
    return (k0, k1, k2, k3), (v0, v1, v2, v3)


def _topk_row(c1b, c2b, r_local, s):
    inf = jnp.float32(jnp.inf)
    init_k = (jnp.full((L,), inf),) * 4
    init_v = (jnp.zeros((L,), jnp.int32),) * 4
    iota = lax.iota(jnp.int32, L)

    def chunk_body(j, carry):
        ks, vs, thresh = carry
        c1v = c1b[r_local, pl.ds(j * L, L)]
        c2v = c2b[r_local, pl.ds(j * L, L)]
        d2 = c1v * c1v + c2v * c2v
        hit = plsc.all_reduce_population_count(d2 < thresh)[0] > 0

        def do_merge(ks, vs, d2):
            ck, cv = plsc.sort_key_val(d2, iota + j * L)
            ks, vs = _merge_chunk(ks, vs, ck, cv)
            return ks, vs, ks[3][15]

        ks, vs, thresh = lax.cond(
            hit, do_merge, lambda ks, vs, d2: (ks, vs, thresh), ks, vs, d2)
        return ks, vs, thresh

    ks, vs, _ = lax.fori_loop(
        0, s // L, chunk_body, (init_k, init_v, jnp.float32(jnp.inf)))
    return ks, vs


def _nn_body(s, coords_hbm, xflat_hbm, idx_hbm, cs1_hbm, cs2_hbm, xout_hbm,
             c1b, c2b, idxb, cs1b, cs2b, shiftb, xgb, sem):
    rows_per_w = (coords_hbm.shape[0] * s) // NW
    wpb = s // rows_per_w
    wid = lax.axis_index("s") * NC + lax.axis_index("c")
    b = wid // wpb
    t_base = (wid % wpb) * rows_per_w

    def group_body(g, _):
        t0 = t_base + g * GROUP
        r0 = b * s + t0
        pltpu.sync_copy(coords_hbm.at[b, 0, pl.ds(t0, GROUP)], c1b)
        pltpu.sync_copy(coords_hbm.at[b, 1, pl.ds(t0, GROUP)], c2b)
        for r in range(GROUP):
            ks, vs = _topk_row(c1b, c2b, r, s)
            rsplat = jnp.full((L,), r, jnp.int32)
            for i in range(4):
                idxb[r, pl.ds(i * L, L)] = vs[i]
                cs1b[r, pl.ds(i * L, L)] = plsc.load_gather(
                    c1b, [rsplat, vs[i]])
                cs2b[r, pl.ds(i * L, L)] = plsc.load_gather(
                    c2b, [rsplat, vs[i]])
                p = r * NH + i * L
                shiftb[p // 128, pl.ds(p % 128, L)] = vs[i] + b * s
        pltpu.sync_copy(idxb, idx_hbm.at[pl.ds(r0, GROUP)])
        pltpu.sync_copy(cs1b, cs1_hbm.at[pl.ds(r0, GROUP)])
        pltpu.sync_copy(cs2b, cs2_hbm.at[pl.ds(r0, GROUP)])
        ngath = (GROUP * NH) // 128
        copies = [
            pltpu.async_copy(
                xflat_hbm.at[shiftb.at[j]],
                xgb.at[pl.ds(j * 128, 128)], sem)
            for j in range(ngath)
        ]
        for c in copies:
            c.wait()
        pltpu.sync_copy(xgb, xout_hbm.at[pl.ds(r0 * NH, GROUP * NH)])
        return 0

    lax.fori_loop(0, rows_per_w // GROUP, group_body, 0)


def kernel(x, coords):
    bsz, s, e = x.shape
    xflat = x.reshape(bsz * s, e)
    mesh = plsc.VectorSubcoreMesh(
        core_axis_name="c", subcore_axis_name="s",
        num_cores=NC, num_subcores=NS)
    run = pl.kernel(
        functools.partial(_nn_body, s),
        out_type=(
            jax.ShapeDtypeStruct((bsz * s, NH), jnp.int32),
            jax.ShapeDtypeStruct((bsz * s, NH), jnp.float32),
            jax.ShapeDtypeStruct((bsz * s, NH), jnp.float32),
            jax.ShapeDtypeStruct((bsz * s * NH, e), jnp.float32),
        ),
        mesh=mesh,
        compiler_params=pltpu.CompilerParams(needs_layout_passes=False),
        scratch_types=[
            pltpu.VMEM((GROUP, s), jnp.float32),
            pltpu.VMEM((GROUP, s), jnp.float32),
            pltpu.VMEM((GROUP, NH), jnp.int32),
            pltpu.VMEM((GROUP, NH), jnp.float32),
            pltpu.VMEM((GROUP, NH), jnp.float32),
            pltpu.VMEM(((GROUP * NH) // 128, 128), jnp.int32),
            pltpu.VMEM((GROUP * NH, e), jnp.float32),
            pltpu.SemaphoreType.DMA,
        ],
    )
    idx, cs1, cs2, xg = run(coords, xflat)
    indices_nh = idx.reshape(bsz, s, NH)
    x_bs = xg.reshape(bsz, s, NH, e)
    cs = jnp.stack(
        [cs1.reshape(bsz, s, NH), cs2.reshape(bsz, s, NH)], axis=1)
    return (x_bs, indices_nh, cs)


# speedup vs baseline: 3.8981x; 1.1811x over previous
import functools

import jax
import jax.numpy as jnp
from jax import lax
from jax.experimental import pallas as pl
from jax.experimental.pallas import tpu as pltpu
from jax.experimental.pallas import tpu_sc as plsc

NH = 64
L = 16
NC, NS = 2, 16
NW = NC * NS
GROUP = 4


def _ce(ka, va, kb, vb):
    c = ka <= kb
    lo_k = jnp.where(c, ka, kb)
    lo_v = jnp.where(c, va, vb)
    hi_k = jnp.where(c, kb, ka)
    hi_v = jnp.where(c, vb, va)
    return lo_k, lo_v, hi_k, hi_v


def _merge_chunk(ks, vs, ck, cv):
    k0, k1, k2, k3 = ks
    v0, v1, v2, v3 = vs
    rk = lax.rev(ck, (0,))
    rv = lax.rev(cv, (0,))
    k3, v3, _, _ = _ce(k3, v3, rk, rv)
    k0, v0, k2, v2 = _ce(k0, v0, k2, v2)
    k1, v1, k3, v3 = _ce(k1, v1, k3, v3)
    k0, v0, k1, v1 = _ce(k0, v0, k1, v1)
    k2, v2, k3, v3 = _ce(k2, v2, k3, v3)
    k0, v0 = plsc.sort_key_val(k0, v0)
    k1, v1 = plsc.sort_key_val(k1, v1)
    k2, v2 = plsc.sort_key_val(k2, v2)
    k3, v3 = plsc.sort_key_val(k3, v3)
    return (k0, k1, k2, k3), (v0, v1, v2, v3)


def _topk_row(c1b, c2b, r_local, s):
    inf = jnp.float32(jnp.inf)
    init_k = (jnp.full((L,), inf),) * 4
    init_v = (jnp.zeros((L,), jnp.int32),) * 4
    iota = lax.iota(jnp.int32, L)

    def chunk_body(j, carry):
        ks, vs, thresh = carry
        c1v = c1b[r_local, pl.ds(j * L, L)]
        c2v = c2b[r_local, pl.ds(j * L, L)]
        d2 = c1v * c1v + c2v * c2v
        hit = plsc.all_reduce_population_count(d2 < thresh)[0] > 0

        def do_merge(ks, vs, d2):
            ck, cv = plsc.sort_key_val(d2, iota + j * L)
            ks, vs = _merge_chunk(ks, vs, ck, cv)
            return ks, vs, ks[3][15]

        ks, vs, thresh = lax.cond(
            hit, do_merge, lambda ks, vs, d2: (ks, vs, thresh), ks, vs, d2)
        return ks, vs, thresh

    ks, vs, _ = lax.fori_loop(
        0, s // L, chunk_body, (init_k, init_v, jnp.float32(jnp.inf)))
    return ks, vs


def _nn_body(s, coords_hbm, xflat_hbm, idx_hbm, cs1_hbm, cs2_hbm, xout_hbm,
             c1b0, c1b1, c2b0, c2b1, idxb0, idxb1, cs1b0, cs1b1,
             cs2b0, cs2b1, shiftb0, shiftb1, xgb0, xgb1,
             isem0, isem1, gsem0, gsem1, osem0, osem1):
    c1b = (c1b0, c1b1)
    c2b = (c2b0, c2b1)
    idxb = (idxb0, idxb1)
    cs1b = (cs1b0, cs1b1)
    cs2b = (cs2b0, cs2b1)
    shiftb = (shiftb0, shiftb1)
    xgb = (xgb0, xgb1)
    isem = (isem0, isem1)
    gsem = (gsem0, gsem1)
    osem = (osem0, osem1)

    rows_per_w = (coords_hbm.shape[0] * s) // NW
    n_groups = rows_per_w // GROUP
    wpb = s // rows_per_w
    wid = lax.axis_index("s") * NC + lax.axis_index("c")
    b = wid // wpb
    t_base = (wid % wpb) * rows_per_w
    ngath = (GROUP * NH) // 128

    def start_in(g, ph):
        t0 = t_base + g * GROUP
        pltpu.async_copy(coords_hbm.at[b, 0, pl.ds(t0, GROUP)],
                         c1b[ph], isem[ph])
        pltpu.async_copy(coords_hbm.at[b, 1, pl.ds(t0, GROUP)],
                         c2b[ph], isem[ph])

    def wait_in(ph):
        pltpu.make_async_copy(coords_hbm.at[b, 0, pl.ds(t_base, GROUP)],
                              c1b[ph], isem[ph]).wait()
        pltpu.make_async_copy(coords_hbm.at[b, 1, pl.ds(t_base, GROUP)],
                              c2b[ph], isem[ph]).wait()

    def start_gather(ph):
        for j in range(ngath):
            pltpu.async_copy(xflat_hbm.at[shiftb[ph].at[j]],
                             xgb[ph].at[pl.ds(j * 128, 128)], gsem[ph])

    def wait_gather(ph):
        for j in range(ngath):
            pltpu.make_async_copy(xflat_hbm.at[shiftb[ph].at[j]],
                                  xgb[ph].at[pl.ds(j * 128, 128)],
                                  gsem[ph]).wait()

    def start_out(g, ph):
        r0 = b * s + t_base + g * GROUP
        pltpu.async_copy(xgb[ph], xout_hbm.at[pl.ds(r0 * NH, GROUP * NH)],
                         osem[ph])

    def wait_out(ph):
        pltpu.make_async_copy(xgb[ph], xout_hbm.at[pl.ds(0, GROUP * NH)],
                              osem[ph]).wait()

    def run_group(g, ph):
        other = 1 - ph
        t0 = t_base + g * GROUP
        r0 = b * s + t0
        for r in range(GROUP):
            ks, vs = _topk_row(c1b[ph], c2b[ph], r, s)
            rsplat = jnp.full((L,), r, jnp.int32)
            for i in range(4):
                idxb[ph][r, pl.ds(i * L, L)] = vs[i]
                cs1b[ph][r, pl.ds(i * L, L)] = plsc.load_gather(
                    c1b[ph], [rsplat, vs[i]])
                cs2b[ph][r, pl.ds(i * L, L)] = plsc.load_gather(
                    c2b[ph], [rsplat, vs[i]])
                p = r * NH + i * L
                shiftb[ph][p // 128, pl.ds(p % 128, L)] = vs[i] + b * s
        pltpu.sync_copy(idxb[ph], idx_hbm.at[pl.ds(r0, GROUP)])
        pltpu.sync_copy(cs1b[ph], cs1_hbm.at[pl.ds(r0, GROUP)])
        pltpu.sync_copy(cs2b[ph], cs2_hbm.at[pl.ds(r0, GROUP)])

    start_in(0, 0)

    def pair_body(g2, _):
        g = 2 * g2
        wait_in(0)
        start_in(g + 1, 1)
        run_group(g, 0)

        @pl.when(g2 >= 1)
        def _():
            wait_out(0)
        start_gather(0)

        @pl.when(g2 >= 1)
        def _():
            wait_gather(1)
            start_out(g - 1, 1)

        g = 2 * g2 + 1
        wait_in(1)

        @pl.when(g2 < n_groups // 2 - 1)
        def _():
            start_in(g + 1, 0)
        run_group(g, 1)

        @pl.when(g2 >= 1)
        def _():
            wait_out(1)
        wait_gather(0)
        start_out(g - 1, 0)
        start_gather(1)
        return 0

    lax.fori_loop(0, n_groups // 2, pair_body, 0)

    wait_gather(1)
    start_out(n_groups - 1, 1)
    wait_out(0)
    wait_out(1)


def kernel(x, coords):
    bsz, s, e = x.shape
    xflat = x.reshape(bsz * s, e)
    mesh = plsc.VectorSubcoreMesh(
        core_axis_name="c", subcore_axis_name="s",
        num_cores=NC, num_subcores=NS)
    run = pl.kernel(
        functools.partial(_nn_body, s),
        out_type=(
            jax.ShapeDtypeStruct((bsz * s, NH), jnp.int32),
            jax.ShapeDtypeStruct((bsz * s, NH), jnp.float32),
            jax.ShapeDtypeStruct((bsz * s, NH), jnp.float32),
            jax.ShapeDtypeStruct((bsz * s * NH, e), jnp.float32),
        ),
        mesh=mesh,
        compiler_params=pltpu.CompilerParams(needs_layout_passes=False),
        scratch_types=[
            pltpu.VMEM((GROUP, s), jnp.float32),
            pltpu.VMEM((GROUP, s), jnp.float32),
            pltpu.VMEM((GROUP, s), jnp.float32),
            pltpu.VMEM((GROUP, s), jnp.float32),
            pltpu.VMEM((GROUP, NH), jnp.int32),
            pltpu.VMEM((GROUP, NH), jnp.int32),
            pltpu.VMEM((GROUP, NH), jnp.float32),
            pltpu.VMEM((GROUP, NH), jnp.float32),
            pltpu.VMEM((GROUP, NH), jnp.float32),
            pltpu.VMEM((GROUP, NH), jnp.float32),
            pltpu.VMEM(((GROUP * NH) // 128, 128), jnp.int32),
            pltpu.VMEM(((GROUP * NH) // 128, 128), jnp.int32),
            pltpu.VMEM((GROUP * NH, e), jnp.float32),
            pltpu.VMEM((GROUP * NH, e), jnp.float32),
            pltpu.SemaphoreType.DMA,
            pltpu.SemaphoreType.DMA,
            pltpu.SemaphoreType.DMA,
            pltpu.SemaphoreType.DMA,
            pltpu.SemaphoreType.DMA,
            pltpu.SemaphoreType.DMA,
        ],
    )
    idx, cs1, cs2, xg = run(coords, xflat)
    indices_nh = idx.reshape(bsz, s, NH)
    x_bs = xg.reshape(bsz, s, NH, e)
    cs = jnp.stack(
        [cs1.reshape(bsz, s, NH), cs2.reshape(bsz, s, NH)], axis=1)
    return (x_bs, indices_nh, cs)


# speedup vs baseline: 5.1559x; 1.3227x over previous
import functools

import jax
import jax.numpy as jnp
from jax import lax
from jax.experimental import pallas as pl
from jax.experimental.pallas import tpu as pltpu
from jax.experimental.pallas import tpu_sc as plsc

NH = 64
L = 16
NC, NS = 2, 16
NW = NC * NS
GROUP = 4
CAP = 48


def _ce(ka, va, kb, vb):
    c = ka <= kb
    lo_k = jnp.where(c, ka, kb)
    lo_v = jnp.where(c, va, vb)
    hi_k = jnp.where(c, kb, ka)
    hi_v = jnp.where(c, vb, va)
    return lo_k, lo_v, hi_k, hi_v


def _rev(k, v):
    return lax.rev(k, (0,)), lax.rev(v, (0,))


def _sort2(ka, va, kb, vb):
    rkb, rvb = _rev(kb, vb)
    lo_k, lo_v, hi_k, hi_v = _ce(ka, va, rkb, rvb)
    lo_k, lo_v = plsc.sort_key_val(lo_k, lo_v)
    hi_k, hi_v = plsc.sort_key_val(hi_k, hi_v)
    return lo_k, lo_v, hi_k, hi_v


def _merge64(ks, vs, cs_k, cs_v):
    lo = []
    for i in range(4):
        rk, rv = _rev(cs_k[3 - i], cs_v[3 - i])
        lk, lv, _, _ = _ce(ks[i], vs[i], rk, rv)
        lo.append((lk, lv))
    (k0, v0), (k1, v1), (k2, v2), (k3, v3) = lo
    k0, v0, k2, v2 = _ce(k0, v0, k2, v2)
    k1, v1, k3, v3 = _ce(k1, v1, k3, v3)
    k0, v0, k1, v1 = _ce(k0, v0, k1, v1)
    k2, v2, k3, v3 = _ce(k2, v2, k3, v3)
    k0, v0 = plsc.sort_key_val(k0, v0)
    k1, v1 = plsc.sort_key_val(k1, v1)
    k2, v2 = plsc.sort_key_val(k2, v2)
    k3, v3 = plsc.sort_key_val(k3, v3)
    return (k0, k1, k2, k3), (v0, v1, v2, v3)


def _sort_buffer(kbuf, vbuf):
    bk = [kbuf[pl.ds(i * L, L)] for i in range(4)]
    bv = [vbuf[pl.ds(i * L, L)] for i in range(4)]
    for i in range(4):
        bk[i], bv[i] = plsc.sort_key_val(bk[i], bv[i])
    bk[0], bv[0], bk[1], bv[1] = _sort2(bk[0], bv[0], bk[1], bv[1])
    bk[2], bv[2], bk[3], bv[3] = _sort2(bk[2], bv[2], bk[3], bv[3])
    rk3, rv3 = _rev(bk[3], bv[3])
    rk2, rv2 = _rev(bk[2], bv[2])
    l0k, l0v, h0k, h0v = _ce(bk[0], bv[0], rk3, rv3)
    l1k, l1v, h1k, h1v = _ce(bk[1], bv[1], rk2, rv2)
    l0k, l0v, l1k, l1v = _ce(l0k, l0v, l1k, l1v)
    h1k, h1v, h0k, h0v = _ce(h1k, h1v, h0k, h0v)
    l0k, l0v = plsc.sort_key_val(l0k, l0v)
    l1k, l1v = plsc.sort_key_val(l1k, l1v)
    h1k, h1v = plsc.sort_key_val(h1k, h1v)
    h0k, h0v = plsc.sort_key_val(h0k, h0v)
    return (l0k, l1k, h1k, h0k), (l0v, l1v, h1v, h0v)


def _topk_row(c1b, c2b, kbuf, vbuf, ktop, vtop, r_local, s):
    inf = jnp.float32(jnp.inf)
    infv = jnp.full((L,), inf)
    iota = lax.iota(jnp.int32, L)
    for i in range(4):
        kbuf[pl.ds(i * L, L)] = infv
        ktop[pl.ds(i * L, L)] = infv

    def rebuild(cnt, thresh):
        bk, bv = _sort_buffer(kbuf, vbuf)
        ks = tuple(ktop[pl.ds(i * L, L)] for i in range(4))
        vs = tuple(vtop[pl.ds(i * L, L)] for i in range(4))
        ks, vs = _merge64(ks, vs, bk, bv)
        for i in range(4):
            ktop[pl.ds(i * L, L)] = ks[i]
            vtop[pl.ds(i * L, L)] = vs[i]
            kbuf[pl.ds(i * L, L)] = infv
        del cnt, thresh
        return jnp.int32(0), ks[3][15]

    def chunk_body(j, carry):
        cnt, thresh = carry
        c1v = c1b[r_local, pl.ds(j * L, L)]
        c2v = c2b[r_local, pl.ds(j * L, L)]
        d2 = c1v * c1v + c2v * c2v
        m = d2 < thresh
        npass = plsc.all_reduce_population_count(m)[0]

        def hit_path(cnt, thresh):
            plsc.store_compressed(kbuf.at[pl.ds(cnt, L)], d2, mask=m)
            plsc.store_compressed(vbuf.at[pl.ds(cnt, L)], iota + j * L, mask=m)
            cnt = cnt + npass
            return lax.cond(cnt >= CAP, rebuild,
                            lambda cnt, thresh: (cnt, thresh), cnt, thresh)

        return lax.cond(npass > 0, hit_path,
                        lambda cnt, thresh: (cnt, thresh), cnt, thresh)

    cnt, _ = lax.fori_loop(0, s // L, chunk_body, (jnp.int32(0), inf))
    lax.cond(cnt > 0, rebuild,
             lambda cnt, thresh: (cnt, thresh), cnt, inf)
    return tuple(vtop[pl.ds(i * L, L)] for i in range(4))


def _nn_body(s, coords_hbm, xflat_hbm, idx_hbm, cs1_hbm, cs2_hbm, xout_hbm,
             c1b0, c1b1, c2b0, c2b1, idxb0, idxb1, cs1b0, cs1b1,
             cs2b0, cs2b1, shiftb0, shiftb1, xgb0, xgb1,
             kbuf, vbuf, ktop, vtop,
             isem0, isem1, gsem0, gsem1, osem0, osem1):
    c1b = (c1b0, c1b1)
    c2b = (c2b0, c2b1)
    idxb = (idxb0, idxb1)
    cs1b = (cs1b0, cs1b1)
    cs2b = (cs2b0, cs2b1)
    shiftb = (shiftb0, shiftb1)
    xgb = (xgb0, xgb1)
    isem = (isem0, isem1)
    gsem = (gsem0, gsem1)
    osem = (osem0, osem1)

    rows_per_w = (coords_hbm.shape[0] * s) // NW
    n_groups = rows_per_w // GROUP
    wpb = s // rows_per_w
    wid = lax.axis_index("s") * NC + lax.axis_index("c")
    b = wid // wpb
    t_base = (wid % wpb) * rows_per_w
    ngath = (GROUP * NH) // 128

    def start_in(g, ph):
        t0 = t_base + g * GROUP
        pltpu.async_copy(coords_hbm.at[b, 0, pl.ds(t0, GROUP)],
                         c1b[ph], isem[ph])
        pltpu.async_copy(coords_hbm.at[b, 1, pl.ds(t0, GROUP)],
                         c2b[ph], isem[ph])

    def wait_in(ph):
        pltpu.make_async_copy(coords_hbm.at[b, 0, pl.ds(t_base, GROUP)],
                              c1b[ph], isem[ph]).wait()
        pltpu.make_async_copy(coords_hbm.at[b, 1, pl.ds(t_base, GROUP)],
                              c2b[ph], isem[ph]).wait()

    def start_gather(ph):
        for j in range(ngath):
            pltpu.async_copy(xflat_hbm.at[shiftb[ph].at[j]],
                             xgb[ph].at[pl.ds(j * 128, 128)], gsem[ph])

    def wait_gather(ph):
        for j in range(ngath):
            pltpu.make_async_copy(xflat_hbm.at[shiftb[ph].at[j]],
                                  xgb[ph].at[pl.ds(j * 128, 128)],
                                  gsem[ph]).wait()

    def start_out(g, ph):
        r0 = b * s + t_base + g * GROUP
        pltpu.async_copy(xgb[ph], xout_hbm.at[pl.ds(r0 * NH, GROUP * NH)],
                         osem[ph])

    def wait_out(ph):
        pltpu.make_async_copy(xgb[ph], xout_hbm.at[pl.ds(0, GROUP * NH)],
                              osem[ph]).wait()

    def run_group(g, ph):
        t0 = t_base + g * GROUP
        r0 = b * s + t0
        for r in range(GROUP):
            vs = _topk_row(c1b[ph], c2b[ph], kbuf, vbuf, ktop, vtop, r, s)
            rsplat = jnp.full((L,), r, jnp.int32)
            for i in range(4):
                idxb[ph][r, pl.ds(i * L, L)] = vs[i]
                cs1b[ph][r, pl.ds(i * L, L)] = plsc.load_gather(
                    c1b[ph], [rsplat, vs[i]])
                cs2b[ph][r, pl.ds(i * L, L)] = plsc.load_gather(
                    c2b[ph], [rsplat, vs[i]])
                p = r * NH + i * L
                shiftb[ph][p // 128, pl.ds(p % 128, L)] = vs[i] + b * s
        pltpu.sync_copy(idxb[ph], idx_hbm.at[pl.ds(r0, GROUP)])
        pltpu.sync_copy(cs1b[ph], cs1_hbm.at[pl.ds(r0, GROUP)])
        pltpu.sync_copy(cs2b[ph], cs2_hbm.at[pl.ds(r0, GROUP)])

    start_in(0, 0)

    def pair_body(g2, _):
        g = 2 * g2
        wait_in(0)
        start_in(g + 1, 1)
        run_group(g, 0)

        @pl.when(g2 >= 1)
        def _():
            wait_out(0)
        start_gather(0)

        @pl.when(g2 >= 1)
        def _():
            wait_gather(1)
            start_out(g - 1, 1)

        g = 2 * g2 + 1
        wait_in(1)

        @pl.when(g2 < n_groups // 2 - 1)
        def _():
            start_in(g + 1, 0)
        run_group(g, 1)

        @pl.when(g2 >= 1)
        def _():
            wait_out(1)
        wait_gather(0)
        start_out(g - 1, 0)
        start_gather(1)
        return 0

    lax.fori_loop(0, n_groups // 2, pair_body, 0)

    wait_gather(1)
    start_out(n_groups - 1, 1)
    wait_out(0)
    wait_out(1)


def kernel(x, coords):
    bsz, s, e = x.shape
    xflat = x.reshape(bsz * s, e)
    mesh = plsc.VectorSubcoreMesh(
        core_axis_name="c", subcore_axis_name="s",
        num_cores=NC, num_subcores=NS)
    run = pl.kernel(
        functools.partial(_nn_body, s),
        out_type=(
            jax.ShapeDtypeStruct((bsz * s, NH), jnp.int32),
            jax.ShapeDtypeStruct((bsz * s, NH), jnp.float32),
            jax.ShapeDtypeStruct((bsz * s, NH), jnp.float32),
            jax.ShapeDtypeStruct((bsz * s * NH, e), jnp.float32),
        ),
        mesh=mesh,
        compiler_params=pltpu.CompilerParams(needs_layout_passes=False),
        scratch_types=[
            pltpu.VMEM((GROUP, s), jnp.float32),
            pltpu.VMEM((GROUP, s), jnp.float32),
            pltpu.VMEM((GROUP, s), jnp.float32),
            pltpu.VMEM((GROUP, s), jnp.float32),
            pltpu.VMEM((GROUP, NH), jnp.int32),
            pltpu.VMEM((GROUP, NH), jnp.int32),
            pltpu.VMEM((GROUP, NH), jnp.float32),
            pltpu.VMEM((GROUP, NH), jnp.float32),
            pltpu.VMEM((GROUP, NH), jnp.float32),
            pltpu.VMEM((GROUP, NH), jnp.float32),
            pltpu.VMEM(((GROUP * NH) // 128, 128), jnp.int32),
            pltpu.VMEM(((GROUP * NH) // 128, 128), jnp.int32),
            pltpu.VMEM((GROUP * NH, e), jnp.float32),
            pltpu.VMEM((GROUP * NH, e), jnp.float32),
            pltpu.VMEM((NH,), jnp.float32),
            pltpu.VMEM((NH,), jnp.int32),
            pltpu.VMEM((NH,), jnp.float32),
            pltpu.VMEM((NH,), jnp.int32),
            pltpu.SemaphoreType.DMA,
            pltpu.SemaphoreType.DMA,
            pltpu.SemaphoreType.DMA,
            pltpu.SemaphoreType.DMA,
            pltpu.SemaphoreType.DMA,
            pltpu.SemaphoreType.DMA,
        ],
    )
    idx, cs1, cs2, xg = run(coords, xflat)
    indices_nh = idx.reshape(bsz, s, NH)
    x_bs = xg.reshape(bsz, s, NH, e)
    cs = jnp.stack(
        [cs1.reshape(bsz, s, NH), cs2.reshape(bsz, s, NH)], axis=1)
    return (x_bs, indices_nh, cs)


# speedup vs baseline: 5.2244x; 1.0133x over previous
import functools

import jax
import jax.numpy as jnp
from jax import lax
from jax.experimental import pallas as pl
from jax.experimental.pallas import tpu as pltpu
from jax.experimental.pallas import tpu_sc as plsc

NH = 64
L = 16
NC, NS = 2, 16
NW = NC * NS
GROUP = 4
CAP = 48


def _ce(ka, va, kb, vb):
    c = ka <= kb
    lo_k = jnp.where(c, ka, kb)
    lo_v = jnp.where(c, va, vb)
    hi_k = jnp.where(c, kb, ka)
    hi_v = jnp.where(c, vb, va)
    return lo_k, lo_v, hi_k, hi_v


def _rev(k, v):
    return lax.rev(k, (0,)), lax.rev(v, (0,))


def _sort2(ka, va, kb, vb):
    rkb, rvb = _rev(kb, vb)
    lo_k, lo_v, hi_k, hi_v = _ce(ka, va, rkb, rvb)
    lo_k, lo_v = plsc.sort_key_val(lo_k, lo_v)
    hi_k, hi_v = plsc.sort_key_val(hi_k, hi_v)
    return lo_k, lo_v, hi_k, hi_v


def _merge64(ks, vs, cs_k, cs_v):
    lo = []
    for i in range(4):
        rk, rv = _rev(cs_k[3 - i], cs_v[3 - i])
        lk, lv, _, _ = _ce(ks[i], vs[i], rk, rv)
        lo.append((lk, lv))
    (k0, v0), (k1, v1), (k2, v2), (k3, v3) = lo
    k0, v0, k2, v2 = _ce(k0, v0, k2, v2)
    k1, v1, k3, v3 = _ce(k1, v1, k3, v3)
    k0, v0, k1, v1 = _ce(k0, v0, k1, v1)
    k2, v2, k3, v3 = _ce(k2, v2, k3, v3)
    k0, v0 = plsc.sort_key_val(k0, v0)
    k1, v1 = plsc.sort_key_val(k1, v1)
    k2, v2 = plsc.sort_key_val(k2, v2)
    k3, v3 = plsc.sort_key_val(k3, v3)
    return (k0, k1, k2, k3), (v0, v1, v2, v3)


def _sort_buffer(kbuf, vbuf):
    bk = [kbuf[pl.ds(i * L, L)] for i in range(4)]
    bv = [vbuf[pl.ds(i * L, L)] for i in range(4)]
    for i in range(4):
        bk[i], bv[i] = plsc.sort_key_val(bk[i], bv[i])
    bk[0], bv[0], bk[1], bv[1] = _sort2(bk[0], bv[0], bk[1], bv[1])
    bk[2], bv[2], bk[3], bv[3] = _sort2(bk[2], bv[2], bk[3], bv[3])
    rk3, rv3 = _rev(bk[3], bv[3])
    rk2, rv2 = _rev(bk[2], bv[2])
    l0k, l0v, h0k, h0v = _ce(bk[0], bv[0], rk3, rv3)
    l1k, l1v, h1k, h1v = _ce(bk[1], bv[1], rk2, rv2)
    l0k, l0v, l1k, l1v = _ce(l0k, l0v, l1k, l1v)
    h1k, h1v, h0k, h0v = _ce(h1k, h1v, h0k, h0v)
    l0k, l0v = plsc.sort_key_val(l0k, l0v)
    l1k, l1v = plsc.sort_key_val(l1k, l1v)
    h1k, h1v = plsc.sort_key_val(h1k, h1v)
    h0k, h0v = plsc.sort_key_val(h0k, h0v)
    return (l0k, l1k, h1k, h0k), (l0v, l1v, h1v, h0v)


def _topk_row(c1b, c2b, kbuf, vbuf, ktop, vtop, r_local, s):
    inf = jnp.float32(jnp.inf)
    infv = jnp.full((L,), inf)
    iota = lax.iota(jnp.int32, L)
    for i in range(4):
        kbuf[pl.ds(i * L, L)] = infv
        ktop[pl.ds(i * L, L)] = infv

    def rebuild(cnt, thresh):
        bk, bv = _sort_buffer(kbuf, vbuf)
        ks = tuple(ktop[pl.ds(i * L, L)] for i in range(4))
        vs = tuple(vtop[pl.ds(i * L, L)] for i in range(4))
        ks, vs = _merge64(ks, vs, bk, bv)
        for i in range(4):
            ktop[pl.ds(i * L, L)] = ks[i]
            vtop[pl.ds(i * L, L)] = vs[i]
            kbuf[pl.ds(i * L, L)] = infv
        del cnt, thresh
        return jnp.int32(0), ks[3][15]

    def chunk_body(j, carry):
        cnt, thresh = carry
        c1v = c1b[r_local, pl.ds(j * L, L)]
        c2v = c2b[r_local, pl.ds(j * L, L)]
        d2 = c1v * c1v + c2v * c2v
        m = d2 < thresh
        npass = plsc.all_reduce_population_count(m)[0]

        def hit_path(cnt, thresh):
            plsc.store_compressed(kbuf.at[pl.ds(cnt, L)], d2, mask=m)
            plsc.store_compressed(vbuf.at[pl.ds(cnt, L)], iota + j * L, mask=m)
            cnt = cnt + npass
            return lax.cond(cnt >= CAP, rebuild,
                            lambda cnt, thresh: (cnt, thresh), cnt, thresh)

        return lax.cond(npass > 0, hit_path,
                        lambda cnt, thresh: (cnt, thresh), cnt, thresh)

    cnt, _ = lax.fori_loop(0, s // L, chunk_body, (jnp.int32(0), inf))
    lax.cond(cnt > 0, rebuild,
             lambda cnt, thresh: (cnt, thresh), cnt, inf)
    return tuple(vtop[pl.ds(i * L, L)] for i in range(4))


def _nn_body(s, coords_hbm, xflat_hbm, idx_hbm, cs1_hbm, cs2_hbm, xout_hbm,
             c1b0, c1b1, c2b0, c2b1, idxb0, idxb1, cs1b0, cs1b1,
             cs2b0, cs2b1, shiftb0, shiftb1, xgb0, xgb1,
             kbuf, vbuf, ktop, vtop,
             isem0, isem1, gsem0, gsem1, osem0, osem1, ssem0, ssem1):
    c1b = (c1b0, c1b1)
    c2b = (c2b0, c2b1)
    idxb = (idxb0, idxb1)
    cs1b = (cs1b0, cs1b1)
    cs2b = (cs2b0, cs2b1)
    shiftb = (shiftb0, shiftb1)
    xgb = (xgb0, xgb1)
    isem = (isem0, isem1)
    gsem = (gsem0, gsem1)
    osem = (osem0, osem1)
    ssem = (ssem0, ssem1)

    rows_per_w = (coords_hbm.shape[0] * s) // NW
    n_groups = rows_per_w // GROUP
    wpb = s // rows_per_w
    wid = lax.axis_index("s") * NC + lax.axis_index("c")
    b = wid // wpb
    t_base = (wid % wpb) * rows_per_w
    ngath = (GROUP * NH) // 128

    def start_in(g, ph):
        t0 = t_base + g * GROUP
        pltpu.async_copy(coords_hbm.at[b, 0, pl.ds(t0, GROUP)],
                         c1b[ph], isem[ph])
        pltpu.async_copy(coords_hbm.at[b, 1, pl.ds(t0, GROUP)],
                         c2b[ph], isem[ph])

    def wait_in(ph):
        pltpu.make_async_copy(coords_hbm.at[b, 0, pl.ds(t_base, GROUP)],
                              c1b[ph], isem[ph]).wait()
        pltpu.make_async_copy(coords_hbm.at[b, 1, pl.ds(t_base, GROUP)],
                              c2b[ph], isem[ph]).wait()

    def start_gather(ph):
        for j in range(ngath):
            pltpu.async_copy(xflat_hbm.at[shiftb[ph].at[j]],
                             xgb[ph].at[pl.ds(j * 128, 128)], gsem[ph])

    def wait_gather(ph):
        for j in range(ngath):
            pltpu.make_async_copy(xflat_hbm.at[shiftb[ph].at[j]],
                                  xgb[ph].at[pl.ds(j * 128, 128)],
                                  gsem[ph]).wait()

    def start_out(g, ph):
        r0 = b * s + t_base + g * GROUP
        pltpu.async_copy(xgb[ph], xout_hbm.at[pl.ds(r0 * NH, GROUP * NH)],
                         osem[ph])

    def wait_out(ph):
        pltpu.make_async_copy(xgb[ph], xout_hbm.at[pl.ds(0, GROUP * NH)],
                              osem[ph]).wait()

    def run_group(g, ph, drain_small):
        t0 = t_base + g * GROUP
        r0 = b * s + t0

        @pl.when(drain_small)
        def _():
            pltpu.make_async_copy(
                idxb[ph], idx_hbm.at[pl.ds(r0, GROUP)], ssem[ph]).wait()
            pltpu.make_async_copy(
                cs1b[ph], cs1_hbm.at[pl.ds(r0, GROUP)], ssem[ph]).wait()
            pltpu.make_async_copy(
                cs2b[ph], cs2_hbm.at[pl.ds(r0, GROUP)], ssem[ph]).wait()
        for r in range(GROUP):
            vs = _topk_row(c1b[ph], c2b[ph], kbuf, vbuf, ktop, vtop, r, s)
            rsplat = jnp.full((L,), r, jnp.int32)
            for i in range(4):
                idxb[ph][r, pl.ds(i * L, L)] = vs[i]
                cs1b[ph][r, pl.ds(i * L, L)] = plsc.load_gather(
                    c1b[ph], [rsplat, vs[i]])
                cs2b[ph][r, pl.ds(i * L, L)] = plsc.load_gather(
                    c2b[ph], [rsplat, vs[i]])
                p = r * NH + i * L
                shiftb[ph][p // 128, pl.ds(p % 128, L)] = vs[i] + b * s
        pltpu.async_copy(idxb[ph], idx_hbm.at[pl.ds(r0, GROUP)], ssem[ph])
        pltpu.async_copy(cs1b[ph], cs1_hbm.at[pl.ds(r0, GROUP)], ssem[ph])
        pltpu.async_copy(cs2b[ph], cs2_hbm.at[pl.ds(r0, GROUP)], ssem[ph])

    start_in(0, 0)

    def pair_body(g2, _):
        g = 2 * g2
        wait_in(0)
        start_in(g + 1, 1)
        run_group(g, 0, g2 >= 1)

        @pl.when(g2 >= 1)
        def _():
            wait_out(0)
        start_gather(0)

        @pl.when(g2 >= 1)
        def _():
            wait_gather(1)
            start_out(g - 1, 1)

        g = 2 * g2 + 1
        wait_in(1)

        @pl.when(g2 < n_groups // 2 - 1)
        def _():
            start_in(g + 1, 0)
        run_group(g, 1, g2 >= 1)

        @pl.when(g2 >= 1)
        def _():
            wait_out(1)
        wait_gather(0)
        start_out(g - 1, 0)
        start_gather(1)
        return 0

    lax.fori_loop(0, n_groups // 2, pair_body, 0)

    wait_gather(1)
    start_out(n_groups - 1, 1)
    wait_out(0)
    wait_out(1)
    for ph in range(2):
        pltpu.make_async_copy(
            idxb[ph], idx_hbm.at[pl.ds(0, GROUP)], ssem[ph]).wait()
        pltpu.make_async_copy(
            cs1b[ph], cs1_hbm.at[pl.ds(0, GROUP)], ssem[ph]).wait()
        pltpu.make_async_copy(
            cs2b[ph], cs2_hbm.at[pl.ds(0, GROUP)], ssem[ph]).wait()


def kernel(x, coords):
    bsz, s, e = x.shape
    xflat = x.reshape(bsz * s, e)
    mesh = plsc.VectorSubcoreMesh(
        core_axis_name="c", subcore_axis_name="s",
        num_cores=NC, num_subcores=NS)
    run = pl.kernel(
        functools.partial(_nn_body, s),
        out_type=(
            jax.ShapeDtypeStruct((bsz * s, NH), jnp.int32),
            jax.ShapeDtypeStruct((bsz * s, NH), jnp.float32),
            jax.ShapeDtypeStruct((bsz * s, NH), jnp.float32),
            jax.ShapeDtypeStruct((bsz * s * NH, e), jnp.float32),
        ),
        mesh=mesh,
        compiler_params=pltpu.CompilerParams(needs_layout_passes=False),
        scratch_types=[
            pltpu.VMEM((GROUP, s), jnp.float32),
            pltpu.VMEM((GROUP, s), jnp.float32),
            pltpu.VMEM((GROUP, s), jnp.float32),
            pltpu.VMEM((GROUP, s), jnp.float32),
            pltpu.VMEM((GROUP, NH), jnp.int32),
            pltpu.VMEM((GROUP, NH), jnp.int32),
            pltpu.VMEM((GROUP, NH), jnp.float32),
            pltpu.VMEM((GROUP, NH), jnp.float32),
            pltpu.VMEM((GROUP, NH), jnp.float32),
            pltpu.VMEM((GROUP, NH), jnp.float32),
            pltpu.VMEM(((GROUP * NH) // 128, 128), jnp.int32),
            pltpu.VMEM(((GROUP * NH) // 128, 128), jnp.int32),
            pltpu.VMEM((GROUP * NH, e), jnp.float32),
            pltpu.VMEM((GROUP * NH, e), jnp.float32),
            pltpu.VMEM((NH,), jnp.float32),
            pltpu.VMEM((NH,), jnp.int32),
            pltpu.VMEM((NH,), jnp.float32),
            pltpu.VMEM((NH,), jnp.int32),
            pltpu.SemaphoreType.DMA,
            pltpu.SemaphoreType.DMA,
            pltpu.SemaphoreType.DMA,
            pltpu.SemaphoreType.DMA,
            pltpu.SemaphoreType.DMA,
            pltpu.SemaphoreType.DMA,
            pltpu.SemaphoreType.DMA,
            pltpu.SemaphoreType.DMA,
        ],
    )
    idx, cs1, cs2, xg = run(coords, xflat)
    indices_nh = idx.reshape(bsz, s, NH)
    x_bs = xg.reshape(bsz, s, NH, e)
    cs = jnp.stack(
        [cs1.reshape(bsz, s, NH), cs2.reshape(bsz, s, NH)], axis=1)
    return (x_bs, indices_nh, cs)
